# baseline (speedup 1.0000x reference)
import functools

import jax
import jax.numpy as jnp
from jax import lax
from jax.experimental import pallas as pl
from jax.experimental.pallas import tpu as pltpu
from jax.experimental.pallas import tpu_sc as plsc

B = 32
V = 1_000_000
NB = 50
CH = 1_250
L = 16
BLK = CH * L
A = 5
UNROLL = 2


@functools.lru_cache(maxsize=1)
def _neg_log_noise():
    noise = jax.random.exponential(jax.random.key(42), (B, V), dtype=jnp.float32)
    nln = -jnp.log(jnp.maximum(noise, 1e-10))
    return nln.reshape(B * V)


def _sampler_body(logits_hbm, temps_hbm, nln_hbm, out_hbm,
                  lbuf0, lbuf1, nbuf0, nbuf1, tbuf, obuf, sem0, sem1):
    wid = lax.axis_index("c") * 16 + lax.axis_index("s")

    pltpu.sync_copy(temps_hbm.at[wid], tbuf)
    tv = tbuf[...]

    bufs = ((lbuf0, nbuf0, sem0), (lbuf1, nbuf1, sem1))

    row0 = wid * (NB * BLK)

    def start(g, b):
        lb, nb, sem = bufs[b]
        off = row0 + g * BLK
        pltpu.async_copy(logits_hbm.at[pl.ds(off, BLK)], lb, sem)
        pltpu.async_copy(nln_hbm.at[pl.ds(off, BLK)], nb, sem)

    def wait(g, b):
        lb, nb, sem = bufs[b]
        off = row0 + g * BLK
        pltpu.make_async_copy(logits_hbm.at[pl.ds(off, BLK)], lb, sem).wait()
        pltpu.make_async_copy(nln_hbm.at[pl.ds(off, BLK)], nb, sem).wait()

    def block(g, b, carry):
        lb, nb, _ = bufs[b]

        def chunks(j, c):
            rs, ids, curs = c
            base = j * (A * L)
            rs, ids, curs = list(rs), list(ids), list(curs)
            for k in range(A):
                off = base + k * L
                v = lb[pl.ds(off, L)] + tv * nb[pl.ds(off, L)]
                m = v > rs[k]
                rs[k] = jnp.where(m, v, rs[k])
                ids[k] = jnp.where(m, curs[k], ids[k])
                curs[k] = curs[k] + A * L
            return tuple(rs), tuple(ids), tuple(curs)

        return lax.fori_loop(0, CH // A, chunks, carry, unroll=UNROLL)

    r0 = tuple(jnp.full((L,), -jnp.inf, dtype=jnp.float32) for _ in range(A))
    i0 = tuple(jnp.zeros((L,), dtype=jnp.int32) for _ in range(A))
    c0 = tuple(lax.iota(jnp.int32, L) + k * L for k in range(A))

    start(0, 0)

    def step(s, carry):
        g0 = 2 * s
        start(g0 + 1, 1)
        wait(g0, 0)
        carry = block(g0, 0, carry)
        start(g0 + 2, 0)
        wait(g0 + 1, 1)
        return block(g0 + 1, 1, carry)

    carry = lax.fori_loop(0, NB // 2 - 1, step, (r0, i0, c0))
    start(NB - 1, 1)
    wait(NB - 2, 0)
    carry = block(NB - 2, 0, carry)
    wait(NB - 1, 1)
    rs, ids, _ = block(NB - 1, 1, carry)

    def merge(a, b):
        ra, ia = a
        rb, ib = b
        m = (rb > ra) | ((rb == ra) & (ib < ia))
        return jnp.where(m, rb, ra), jnp.where(m, ib, ia)

    pairs = list(zip(rs, ids))
    while len(pairs) > 1:
        nxt = [merge(pairs[i], pairs[i + 1]) for i in range(0, len(pairs) - 1, 2)]
        if len(pairs) % 2:
            nxt.append(pairs[-1])
        pairs = nxt
    r, bidx = pairs[0]

    bv, bi = r[0], bidx[0]
    for i in range(1, L):
        rv, iv = r[i], bidx[i]
        better = (rv > bv) | ((rv == bv) & (iv < bi))
        bv = jnp.where(better, rv, bv)
        bi = jnp.where(better, iv, bi)

    obuf[...] = jnp.full((L,), bi, dtype=jnp.int32)
    pltpu.sync_copy(obuf, out_hbm.at[wid])


_sampler = pl.kernel(
    _sampler_body,
    out_type=jax.ShapeDtypeStruct((B, L), jnp.int32),
    mesh=plsc.VectorSubcoreMesh(core_axis_name="c", subcore_axis_name="s"),
    scratch_types=[
        pltpu.VMEM((BLK,), jnp.float32),
        pltpu.VMEM((BLK,), jnp.float32),
        pltpu.VMEM((BLK,), jnp.float32),
        pltpu.VMEM((BLK,), jnp.float32),
        pltpu.VMEM((L,), jnp.float32),
        pltpu.VMEM((L,), jnp.int32),
        pltpu.SemaphoreType.DMA,
        pltpu.SemaphoreType.DMA,
    ],
)


def kernel(logits, temperatures):
    logits1 = logits.reshape(B * V)
    temps2 = jnp.broadcast_to(temperatures[:, None], (B, L))
    out2 = _sampler(logits1, temps2, _neg_log_noise())
    return out2[:, 0]


# speedup vs baseline: 10.7810x; 10.7810x over previous
import functools

import jax
import jax.numpy as jnp
from jax import lax
from jax.experimental import pallas as pl
from jax.experimental.pallas import tpu as pltpu
from jax.experimental.pallas import tpu_sc as plsc

B = 32
V = 1_000_000
NB = 50
CH = 1_250
L = 16
BLK = CH * L
A = 5
UNROLL = 2


@functools.lru_cache(maxsize=1)
def _neg_log_noise():
    noise = jax.random.exponential(jax.random.key(42), (B, V), dtype=jnp.float32)
    nln = -jnp.log(jnp.maximum(noise, 1e-10))
    return nln.reshape(B, NB, BLK)


def _sampler_body(logits_hbm, temps_hbm, nln_hbm, out_hbm,
                  lbuf0, lbuf1, nbuf0, nbuf1, tbuf, obuf, sem0, sem1):
    wid = lax.axis_index("c") * 16 + lax.axis_index("s")

    pltpu.sync_copy(temps_hbm.at[wid], tbuf)
    tv = tbuf[...]

    bufs = ((lbuf0, nbuf0, sem0), (lbuf1, nbuf1, sem1))

    def start(g, b):
        lb, nb, sem = bufs[b]
        pltpu.async_copy(logits_hbm.at[wid, g], lb, sem)
        pltpu.async_copy(nln_hbm.at[wid, g], nb, sem)

    def wait(g, b):
        lb, nb, sem = bufs[b]
        pltpu.make_async_copy(logits_hbm.at[wid, g], lb, sem).wait()
        pltpu.make_async_copy(nln_hbm.at[wid, g], nb, sem).wait()

    def block(g, b, carry):
        lb, nb, _ = bufs[b]

        def chunks(j, c):
            rs, ids, curs = c
            base = j * (A * L)
            rs, ids, curs = list(rs), list(ids), list(curs)
            for k in range(A):
                off = base + k * L
                v = lb[pl.ds(off, L)] + tv * nb[pl.ds(off, L)]
                m = v > rs[k]
                rs[k] = jnp.where(m, v, rs[k])
                ids[k] = jnp.where(m, curs[k], ids[k])
                curs[k] = curs[k] + A * L
            return tuple(rs), tuple(ids), tuple(curs)

        return lax.fori_loop(0, CH // A, chunks, carry, unroll=UNROLL)

    r0 = tuple(jnp.full((L,), -jnp.inf, dtype=jnp.float32) for _ in range(A))
    i0 = tuple(jnp.zeros((L,), dtype=jnp.int32) for _ in range(A))
    c0 = tuple(lax.iota(jnp.int32, L) + k * L for k in range(A))

    start(0, 0)

    def step(s, carry):
        g0 = 2 * s
        start(g0 + 1, 1)
        wait(g0, 0)
        carry = block(g0, 0, carry)
        start(g0 + 2, 0)
        wait(g0 + 1, 1)
        return block(g0 + 1, 1, carry)

    carry = lax.fori_loop(0, NB // 2 - 1, step, (r0, i0, c0))
    start(NB - 1, 1)
    wait(NB - 2, 0)
    carry = block(NB - 2, 0, carry)
    wait(NB - 1, 1)
    rs, ids, _ = block(NB - 1, 1, carry)

    def merge(a, b):
        ra, ia = a
        rb, ib = b
        m = (rb > ra) | ((rb == ra) & (ib < ia))
        return jnp.where(m, rb, ra), jnp.where(m, ib, ia)

    pairs = list(zip(rs, ids))
    while len(pairs) > 1:
        nxt = [merge(pairs[i], pairs[i + 1]) for i in range(0, len(pairs) - 1, 2)]
        if len(pairs) % 2:
            nxt.append(pairs[-1])
        pairs = nxt
    r, bidx = pairs[0]

    bv, bi = r[0], bidx[0]
    for i in range(1, L):
        rv, iv = r[i], bidx[i]
        better = (rv > bv) | ((rv == bv) & (iv < bi))
        bv = jnp.where(better, rv, bv)
        bi = jnp.where(better, iv, bi)

    obuf[...] = jnp.full((L,), bi, dtype=jnp.int32)
    pltpu.sync_copy(obuf, out_hbm.at[wid])


_sampler = pl.kernel(
    _sampler_body,
    out_type=jax.ShapeDtypeStruct((B, L), jnp.int32),
    mesh=plsc.VectorSubcoreMesh(core_axis_name="c", subcore_axis_name="s"),
    scratch_types=[
        pltpu.VMEM((BLK,), jnp.float32),
        pltpu.VMEM((BLK,), jnp.float32),
        pltpu.VMEM((BLK,), jnp.float32),
        pltpu.VMEM((BLK,), jnp.float32),
        pltpu.VMEM((L,), jnp.float32),
        pltpu.VMEM((L,), jnp.int32),
        pltpu.SemaphoreType.DMA,
        pltpu.SemaphoreType.DMA,
    ],
)


def kernel(logits, temperatures):
    logits3 = logits.reshape(B, NB, BLK)
    temps2 = jnp.broadcast_to(temperatures[:, None], (B, L))
    out2 = _sampler(logits3, temps2, _neg_log_noise())
    return out2[:, 0]


# speedup vs baseline: 54.0924x; 5.0174x over previous
import functools

import jax
import jax.numpy as jnp
from jax import lax
from jax.experimental import pallas as pl
from jax.experimental.pallas import tpu as pltpu
from jax.experimental.pallas import tpu_sc as plsc

B = 32
V = 1_000_000
L = 16
W = 160 * 128
NWF = 48
TAILW = 132 * 128
TAIL_CHUNKS = TAILW // L
REM = V - NWF * W - TAILW
A = 4
UNROLL = 2


@functools.lru_cache(maxsize=1)
def _neg_log_noise():
    noise = jax.random.exponential(jax.random.key(42), (B, V), dtype=jnp.float32)
    nln = -jnp.log(jnp.maximum(noise, 1e-10))
    return nln.reshape(4, 8, V), jnp.asarray(nln[:, V - REM:])


_NLN, _NLN_TAIL = _neg_log_noise()


def _sampler_body(logits_hbm, temps_hbm, nln_hbm, ltail_hbm, ntail_hbm, out_hbm,
                  lbuf0, lbuf1, nbuf0, nbuf1, tlb, tnb, tbuf, obuf, sem0, sem1):
    wid = lax.axis_index("c") * 16 + lax.axis_index("s")
    rg = wid // 8
    sl = wid % 8

    pltpu.sync_copy(temps_hbm.at[wid], tbuf)
    tv = tbuf[...]

    bufs = ((lbuf0, nbuf0, sem0), (lbuf1, nbuf1, sem1))

    def start(k, b, size=W):
        lb, nb, sem = bufs[b]
        off = k * W
        pltpu.async_copy(logits_hbm.at[rg, sl, pl.ds(off, size)],
                         lb.at[pl.ds(0, size)], sem)
        pltpu.async_copy(nln_hbm.at[rg, sl, pl.ds(off, size)],
                         nb.at[pl.ds(0, size)], sem)

    def wait(k, b, size=W):
        lb, nb, sem = bufs[b]
        off = k * W
        pltpu.make_async_copy(logits_hbm.at[rg, sl, pl.ds(off, size)],
                              lb.at[pl.ds(0, size)], sem).wait()
        pltpu.make_async_copy(nln_hbm.at[rg, sl, pl.ds(off, size)],
                              nb.at[pl.ds(0, size)], sem).wait()

    def block(b, carry, nch=W // L):
        lb, nb, _ = bufs[b]

        def chunks(j, c):
            rs, ids, curs = c
            base = j * (A * L)
            rs, ids, curs = list(rs), list(ids), list(curs)
            for k in range(A):
                off = base + k * L
                v = lb[pl.ds(off, L)] + tv * nb[pl.ds(off, L)]
                m = v > rs[k]
                rs[k] = jnp.where(m, v, rs[k])
                ids[k] = jnp.where(m, curs[k], ids[k])
                curs[k] = curs[k] + A * L
            return tuple(rs), tuple(ids), tuple(curs)

        return lax.fori_loop(0, nch // A, chunks, carry, unroll=UNROLL)

    r0 = tuple(jnp.full((L,), -jnp.inf, dtype=jnp.float32) for _ in range(A))
    i0 = tuple(jnp.zeros((L,), dtype=jnp.int32) for _ in range(A))
    c0 = tuple(lax.iota(jnp.int32, L) + k * L for k in range(A))

    start(0, 0)

    def step(s, carry):
        k0 = 2 * s
        start(k0 + 1, 1)
        wait(k0, 0)
        carry = block(0, carry)
        start(k0 + 2, 0)
        wait(k0 + 1, 1)
        return block(1, carry)

    carry = lax.fori_loop(0, 23, step, (r0, i0, c0))
    start(47, 1)
    wait(46, 0)
    carry = block(0, carry)
    start(48, 0, size=TAILW)
    wait(47, 1)
    carry = block(1, carry)
    wait(48, 0, size=TAILW)
    rs, ids, _ = block(0, carry, nch=TAIL_CHUNKS)
    rs, ids = list(rs), list(ids)

    pltpu.sync_copy(ltail_hbm.at[wid], tlb)
    pltpu.sync_copy(ntail_hbm.at[wid], tnb)
    for c in range(REM // L):
        v = tlb[pl.ds(c * L, L)] + tv * tnb[pl.ds(c * L, L)]
        cur = lax.iota(jnp.int32, L) + (V - REM + c * L)
        m = v > rs[c]
        rs[c] = jnp.where(m, v, rs[c])
        ids[c] = jnp.where(m, cur, ids[c])

    def merge(a, b):
        ra, ia = a
        rb, ib = b
        m = (rb > ra) | ((rb == ra) & (ib < ia))
        return jnp.where(m, rb, ra), jnp.where(m, ib, ia)

    pairs = list(zip(rs, ids))
    while len(pairs) > 1:
        nxt = [merge(pairs[i], pairs[i + 1]) for i in range(0, len(pairs) - 1, 2)]
        if len(pairs) % 2:
            nxt.append(pairs[-1])
        pairs = nxt
    r, bidx = pairs[0]

    bv, bi = r[0], bidx[0]
    for i in range(1, L):
        rv, iv = r[i], bidx[i]
        better = (rv > bv) | ((rv == bv) & (iv < bi))
        bv = jnp.where(better, rv, bv)
        bi = jnp.where(better, iv, bi)

    obuf[...] = jnp.full((L,), bi, dtype=jnp.int32)
    pltpu.sync_copy(obuf, out_hbm.at[wid])


_sampler = pl.kernel(
    _sampler_body,
    out_type=jax.ShapeDtypeStruct((B, L), jnp.int32),
    mesh=plsc.VectorSubcoreMesh(core_axis_name="c", subcore_axis_name="s"),
    scratch_types=[
        pltpu.VMEM((W,), jnp.float32),
        pltpu.VMEM((W,), jnp.float32),
        pltpu.VMEM((W,), jnp.float32),
        pltpu.VMEM((W,), jnp.float32),
        pltpu.VMEM((REM,), jnp.float32),
        pltpu.VMEM((REM,), jnp.float32),
        pltpu.VMEM((L,), jnp.float32),
        pltpu.VMEM((L,), jnp.int32),
        pltpu.SemaphoreType.DMA,
        pltpu.SemaphoreType.DMA,
    ],
)


def kernel(logits, temperatures):
    logits3 = logits.reshape(4, 8, V)
    ltail = logits[:, V - REM:]
    temps2 = jnp.broadcast_to(temperatures[:, None], (B, L))
    out2 = _sampler(logits3, temps2, _NLN, ltail, _NLN_TAIL)
    return out2[:, 0]
